# Initial kernel scaffold; baseline (speedup 1.0000x reference)
import jax
import jax.numpy as jnp
from jax.experimental import pallas as pl


def kernel(item_id, item_name_tokens, item_gics, emb_id, emb_name, emb_gics):
    raise NotImplementedError("write your pallas kernel here")


# speedup vs baseline: 12.0107x; 12.0107x over previous
import functools

import jax
import jax.numpy as jnp
from jax import lax
from jax.experimental import pallas as pl
from jax.experimental.pallas import tpu as pltpu
from jax.experimental.pallas import tpu_sc as plsc

B = 16384
L = 20
DN = 16
DI = 8
DO = 32
C = 64
CL = C * L


def _build(nc, ns):
    nw = nc * ns
    bw = B // nw
    nchunk = bw // C
    mesh = plsc.VectorSubcoreMesh(core_axis_name="c", subcore_axis_name="s")

    @functools.partial(
        pl.kernel,
        out_type=jax.ShapeDtypeStruct((B * DO,), jnp.float32),
        mesh=mesh,
        compiler_params=pltpu.CompilerParams(
            needs_layout_passes=False, use_tc_tiling_on_sc=False),
        scratch_types=[
            pltpu.VMEM((bw,), jnp.int32),
            pltpu.VMEM((bw,), jnp.int32),
            pltpu.VMEM((bw * L,), jnp.int32),
            pltpu.VMEM((bw, DI), jnp.float32),
            pltpu.VMEM((bw, DI), jnp.float32),
            pltpu.VMEM((2, CL, DN), jnp.float32),
            pltpu.VMEM((bw * DO,), jnp.float32),
            pltpu.VMEM((bw,), jnp.float32),
            pltpu.SemaphoreType.DMA,
            pltpu.SemaphoreType.DMA,
            pltpu.SemaphoreType.DMA,
            pltpu.SemaphoreType.DMA,
        ],
    )
    def body(id_hbm, tok_hbm, gics_hbm, embid_hbm, embname_hbm, embgics_hbm,
             out_hbm, id_idx, gics_idx, tok_idx, id_rows, gics_rows,
             namebuf, out_v, inv_v, sem_id, sem_g, sem_n0, sem_n1):
        wid = lax.axis_index("s") * nc + lax.axis_index("c")
        base = wid * bw

        pltpu.sync_copy(id_hbm.at[pl.ds(base, bw)], id_idx)
        pltpu.sync_copy(gics_hbm.at[pl.ds(base, bw)], gics_idx)
        pltpu.sync_copy(tok_hbm.at[pl.ds(base * L, bw * L)], tok_idx)

        cp_id = pltpu.async_copy(embid_hbm.at[id_idx], id_rows, sem_id)
        cp_g = pltpu.async_copy(embgics_hbm.at[gics_idx], gics_rows, sem_g)

        sems = [sem_n0, sem_n1]

        def start_chunk(c):
            return pltpu.async_copy(
                embname_hbm.at[tok_idx.at[pl.ds(c * CL, CL)]],
                namebuf.at[c % 2], sems[c % 2])

        cps = [start_chunk(0), None]

        iota = lax.iota(jnp.int32, 16)
        iota_l = iota * L

        def cnt_body(g, carry):
            flat0 = g * (16 * L) + iota_l
            cnt = jnp.zeros((16,), jnp.float32)
            for t in range(L):
                tk = plsc.load_gather(tok_idx, [flat0 + t])
                cnt = cnt + jnp.where(tk != 0, 1.0, 0.0)
            inv_v[pl.ds(g * 16, 16)] = 1.0 / jnp.maximum(cnt, 1.0)
            return carry

        lax.fori_loop(0, bw // 16, cnt_body, 0)

        for c in range(nchunk):
            slot = c % 2
            if c + 1 < nchunk:
                cps[(c + 1) % 2] = start_chunk(c + 1)
            cps[slot].wait()
            nb = namebuf.at[slot]

            def row_body(r, carry, c=c, nb=nb):
                row = c * C + r
                vals = [nb[r * L + t, :] for t in range(L)]
                while len(vals) > 1:
                    nxt = [vals[i] + vals[i + 1]
                           for i in range(0, len(vals) - 1, 2)]
                    if len(vals) % 2:
                        nxt.append(vals[-1])
                    vals = nxt
                s = plsc.load_gather(inv_v, [jnp.zeros((16,), jnp.int32) + row])
                out_v[pl.ds(row * DO + DI, DN)] = vals[0] * s
                return carry

            lax.fori_loop(0, C, row_body, 0)

        cp_id.wait()
        cp_g.wait()
        row2 = lax.shift_right_logical(iota, 3)
        col8 = jnp.bitwise_and(iota, 7)
        sc0 = jnp.where(iota < 8, iota, iota + (DO - DI))

        def asm_body(r2, carry):
            off = r2 * (2 * DO)
            ridx = r2 * 2 + row2
            vi = plsc.load_gather(id_rows, [ridx, col8])
            plsc.store_scatter(out_v, [sc0 + off], vi)
            vg = plsc.load_gather(gics_rows, [ridx, col8])
            plsc.store_scatter(out_v, [sc0 + (off + (DO - DI))], vg)
            return carry

        lax.fori_loop(0, bw // 2, asm_body, 0)

        pltpu.sync_copy(out_v, out_hbm.at[pl.ds(base * DO, bw * DO)])

    return body


def kernel(item_id, item_name_tokens, item_gics, emb_id, emb_name, emb_gics):
    info = plsc.get_sparse_core_info()
    body = _build(info.num_cores, info.num_subcores)
    emb_name = emb_name.at[0].set(0.0)
    out = body(item_id.astype(jnp.int32),
               item_name_tokens.astype(jnp.int32).reshape(-1),
               item_gics.astype(jnp.int32),
               emb_id, emb_name, emb_gics)
    return out.reshape(B, DO)


# speedup vs baseline: 19.4186x; 1.6168x over previous
import functools

import jax
import jax.numpy as jnp
from jax import lax
from jax.experimental import pallas as pl
from jax.experimental.pallas import tpu as pltpu
from jax.experimental.pallas import tpu_sc as plsc

B = 16384
L = 20
DN = 16
DI = 8
DO = 32
C = 64
CL = C * L


def _build(nc, ns):
    nw = nc * ns
    bw = B // nw
    nchunk = bw // C
    mesh = plsc.VectorSubcoreMesh(core_axis_name="c", subcore_axis_name="s")

    @functools.partial(
        pl.kernel,
        out_type=jax.ShapeDtypeStruct((DO, B), jnp.float32),
        mesh=mesh,
        compiler_params=pltpu.CompilerParams(
            needs_layout_passes=False, use_tc_tiling_on_sc=False),
        scratch_types=[
            pltpu.VMEM((bw,), jnp.int32),
            pltpu.VMEM((bw,), jnp.int32),
            pltpu.VMEM((L, bw), jnp.int32),
            pltpu.VMEM((bw * L,), jnp.int32),
            pltpu.VMEM((DI, bw), jnp.float32),
            pltpu.VMEM((DI, bw), jnp.float32),
            pltpu.VMEM((2, CL, DN), jnp.float32),
            pltpu.VMEM((DN, bw), jnp.float32),
            pltpu.VMEM((bw,), jnp.float32),
            pltpu.SemaphoreType.DMA,
            pltpu.SemaphoreType.DMA,
            pltpu.SemaphoreType.DMA,
            pltpu.SemaphoreType.DMA,
        ],
    )
    def body(id_hbm, tokt_hbm, gics_hbm, embid_hbm, embname_hbm, embgics_hbm,
             out_hbm, id_idx, gics_idx, tok_t, tok_b, id_rows, gics_rows,
             namebuf, pooled, inv_v, sem_id, sem_g, sem_n0, sem_n1):
        wid = lax.axis_index("s") * nc + lax.axis_index("c")
        base = wid * bw

        pltpu.sync_copy(id_hbm.at[pl.ds(base, bw)], id_idx)
        pltpu.sync_copy(gics_hbm.at[pl.ds(base, bw)], gics_idx)
        pltpu.sync_copy(tokt_hbm.at[:, pl.ds(base, bw)], tok_t)

        cps_idg = []
        for d in range(DI):
            cps_idg.append(pltpu.async_copy(
                embid_hbm.at[d].at[id_idx], id_rows.at[d], sem_id))
            cps_idg.append(pltpu.async_copy(
                embgics_hbm.at[d].at[gics_idx], gics_rows.at[d], sem_g))

        iota = lax.iota(jnp.int32, 16)

        def cnt_body(g, carry):
            cnt = jnp.zeros((16,), jnp.float32)
            for t in range(L):
                tk = tok_t[t, pl.ds(g * 16, 16)]
                cnt = cnt + jnp.where(tk != 0, 1.0, 0.0)
            inv_v[pl.ds(g * 16, 16)] = 1.0 / jnp.maximum(cnt, 1.0)
            return carry

        lax.fori_loop(0, bw // 16, cnt_body, 0)

        iota_l = iota * L

        def trans_body(g, carry):
            gbase = g * 16
            for t in range(L):
                v = tok_t[t, pl.ds(gbase, 16)]
                plsc.store_scatter(tok_b, [gbase * L + iota_l + t], v)
            return carry

        lax.fori_loop(0, bw // 16, trans_body, 0)

        sems = [sem_n0, sem_n1]

        def start_chunk(c):
            return pltpu.async_copy(
                embname_hbm.at[tok_b.at[pl.ds(c * CL, CL)]],
                namebuf.at[c % 2], sems[c % 2])

        cps = [start_chunk(0), None]

        def make_row_body(c, nb):
            def row_body(r, carry):
                row = c * C + r
                vals = [nb[r * L + t, :] for t in range(L)]
                while len(vals) > 1:
                    nxt = [vals[i] + vals[i + 1]
                           for i in range(0, len(vals) - 1, 2)]
                    if len(vals) % 2:
                        nxt.append(vals[-1])
                    vals = nxt
                s = plsc.load_gather(inv_v, [jnp.zeros((16,), jnp.int32) + row])
                plsc.store_scatter(pooled,
                                   [iota, jnp.zeros((16,), jnp.int32) + row],
                                   vals[0] * s)
                return carry
            return row_body

        for c in range(nchunk):
            slot = c % 2
            if c + 1 < nchunk:
                cps[(c + 1) % 2] = start_chunk(c + 1)
            cps[slot].wait()
            lax.fori_loop(0, C, make_row_body(c, namebuf.at[slot]), 0)

        for cp in cps_idg:
            cp.wait()
        pltpu.sync_copy(id_rows, out_hbm.at[pl.ds(0, DI), pl.ds(base, bw)])
        pltpu.sync_copy(pooled, out_hbm.at[pl.ds(DI, DN), pl.ds(base, bw)])
        pltpu.sync_copy(gics_rows,
                        out_hbm.at[pl.ds(DI + DN, DI), pl.ds(base, bw)])

    return body


def kernel(item_id, item_name_tokens, item_gics, emb_id, emb_name, emb_gics):
    info = plsc.get_sparse_core_info()
    body = _build(info.num_cores, info.num_subcores)
    emb_name = emb_name.at[0].set(0.0)
    out_t = body(item_id.astype(jnp.int32),
                 item_name_tokens.astype(jnp.int32).T,
                 item_gics.astype(jnp.int32),
                 emb_id.T, emb_name, emb_gics.T)
    return out_t.T


# speedup vs baseline: 25.9060x; 1.3341x over previous
import functools

import jax
import jax.numpy as jnp
from jax import lax
from jax.experimental import pallas as pl
from jax.experimental.pallas import tpu as pltpu
from jax.experimental.pallas import tpu_sc as plsc

B = 16384
L = 20
DN = 16
DI = 8
DO = 32
VID = 100001
VG = 1001
C = 64
CL = C * L


def _build(nc, ns):
    nw = nc * ns
    bw = B // nw
    nchunk = bw // C
    mesh = plsc.VectorSubcoreMesh(core_axis_name="c", subcore_axis_name="s")

    @functools.partial(
        pl.kernel,
        out_type=jax.ShapeDtypeStruct((DO, B), jnp.float32),
        mesh=mesh,
        compiler_params=pltpu.CompilerParams(
            needs_layout_passes=False, use_tc_tiling_on_sc=False),
        scratch_types=[
            pltpu.VMEM((bw,), jnp.int32),
            pltpu.VMEM((DI * bw,), jnp.int32),
            pltpu.VMEM((bw,), jnp.int32),
            pltpu.VMEM((L, bw), jnp.int32),
            pltpu.VMEM((bw * L,), jnp.int32),
            pltpu.VMEM((DI * bw,), jnp.float32),
            pltpu.VMEM((DI, bw), jnp.float32),
            pltpu.VMEM((DI * VG,), jnp.float32),
            pltpu.VMEM((2, CL, DN), jnp.float32),
            pltpu.VMEM((DN, bw), jnp.float32),
            pltpu.VMEM((bw,), jnp.float32),
            pltpu.SemaphoreType.DMA,
            pltpu.SemaphoreType.DMA,
            pltpu.SemaphoreType.DMA,
            pltpu.SemaphoreType.DMA,
        ],
    )
    def body(id_hbm, tokt_hbm, gics_hbm, embid_hbm, embname_hbm, embgics_hbm,
             out_hbm, id_idx, id_idx8, gics_idx, tok_t, tok_b, id_rows,
             gics_rows, gics_tab, namebuf, pooled, inv_v,
             sem_id, sem_g, sem_n0, sem_n1):
        wid = lax.axis_index("s") * nc + lax.axis_index("c")
        base = wid * bw

        pltpu.sync_copy(id_hbm.at[pl.ds(base, bw)], id_idx)
        pltpu.sync_copy(gics_hbm.at[pl.ds(base, bw)], gics_idx)
        pltpu.sync_copy(tokt_hbm.at[:, pl.ds(base, bw)], tok_t)

        iota = lax.iota(jnp.int32, 16)
        iota_l = iota * L

        def trans_body(g, carry):
            gbase = g * 16
            for t in range(L):
                v = tok_t[t, pl.ds(gbase, 16)]
                plsc.store_scatter(tok_b, [gbase * L + iota_l + t], v)
            return carry

        lax.fori_loop(0, bw // 16, trans_body, 0)

        sems = [sem_n0, sem_n1]

        def start_chunk(c):
            return pltpu.async_copy(
                embname_hbm.at[tok_b.at[pl.ds(c * CL, CL)]],
                namebuf.at[c % 2], sems[c % 2])

        cps = [start_chunk(0), start_chunk(1)]

        def idx8_body(g, carry):
            v = id_idx[pl.ds(g * 16, 16)]
            for d in range(DI):
                id_idx8[pl.ds(d * bw + g * 16, 16)] = v + d * VID
            return carry

        lax.fori_loop(0, bw // 16, idx8_body, 0)
        cp_id = pltpu.async_copy(embid_hbm.at[id_idx8], id_rows, sem_id)
        cp_gt = pltpu.async_copy(embgics_hbm, gics_tab, sem_g)

        def cnt_body(g, carry):
            cnt = jnp.zeros((16,), jnp.float32)
            for t in range(L):
                tk = tok_t[t, pl.ds(g * 16, 16)]
                cnt = cnt + jnp.where(tk != 0, 1.0, 0.0)
            inv_v[pl.ds(g * 16, 16)] = 1.0 / jnp.maximum(cnt, 1.0)
            return carry

        lax.fori_loop(0, bw // 16, cnt_body, 0)

        def make_row_body(c, nb):
            def row_body(r, carry):
                row = c * C + r
                vals = [nb[r * L + t, :] for t in range(L)]
                while len(vals) > 1:
                    nxt = [vals[i] + vals[i + 1]
                           for i in range(0, len(vals) - 1, 2)]
                    if len(vals) % 2:
                        nxt.append(vals[-1])
                    vals = nxt
                s = plsc.load_gather(inv_v, [jnp.zeros((16,), jnp.int32) + row])
                plsc.store_scatter(pooled,
                                   [iota, jnp.zeros((16,), jnp.int32) + row],
                                   vals[0] * s)
                return carry
            return row_body

        for c in range(nchunk):
            slot = c % 2
            if c + 2 < nchunk:
                cps[slot].wait()
                lax.fori_loop(0, C, make_row_body(c, namebuf.at[slot]), 0)
                cps[slot] = start_chunk(c + 2)
            else:
                cps[slot].wait()
                lax.fori_loop(0, C, make_row_body(c, namebuf.at[slot]), 0)

        cp_gt.wait()

        def gics_body(g, carry):
            gv = gics_idx[pl.ds(g * 16, 16)]
            for d in range(DI):
                r = plsc.load_gather(gics_tab, [gv + d * VG])
                gics_rows[d, pl.ds(g * 16, 16)] = r
            return carry

        lax.fori_loop(0, bw // 16, gics_body, 0)

        cp_id.wait()
        outs = [pltpu.async_copy(id_rows.at[pl.ds(d * bw, bw)],
                                 out_hbm.at[d, pl.ds(base, bw)], sem_id)
                for d in range(DI)]
        outs.append(pltpu.async_copy(
            pooled, out_hbm.at[pl.ds(DI, DN), pl.ds(base, bw)], sem_n0))
        outs.append(pltpu.async_copy(
            gics_rows, out_hbm.at[pl.ds(DI + DN, DI), pl.ds(base, bw)],
            sem_g))
        for cp in outs:
            cp.wait()

    return body


def kernel(item_id, item_name_tokens, item_gics, emb_id, emb_name, emb_gics):
    info = plsc.get_sparse_core_info()
    body = _build(info.num_cores, info.num_subcores)
    emb_name = emb_name.at[0].set(0.0)
    out_t = body(item_id.astype(jnp.int32),
                 item_name_tokens.astype(jnp.int32).T,
                 item_gics.astype(jnp.int32),
                 emb_id.T.reshape(-1), emb_name, emb_gics.T.reshape(-1))
    return out_t.T


# speedup vs baseline: 27.1298x; 1.0472x over previous
import functools

import jax
import jax.numpy as jnp
from jax import lax
from jax.experimental import pallas as pl
from jax.experimental.pallas import tpu as pltpu
from jax.experimental.pallas import tpu_sc as plsc

B = 16384
L = 20
DN = 16
DI = 8
DO = 32
VID = 100001
VG = 1001
C = 64
CL = C * L


def _build(nc, ns):
    nw = nc * ns
    bw = B // nw
    nchunk = bw // C
    mesh = plsc.VectorSubcoreMesh(core_axis_name="c", subcore_axis_name="s")

    @functools.partial(
        pl.kernel,
        out_type=jax.ShapeDtypeStruct((DO, B), jnp.float32),
        mesh=mesh,
        compiler_params=pltpu.CompilerParams(
            needs_layout_passes=False, use_tc_tiling_on_sc=False),
        scratch_types=[
            pltpu.VMEM((bw,), jnp.int32),
            pltpu.VMEM((DI * bw,), jnp.int32),
            pltpu.VMEM((bw,), jnp.int32),
            pltpu.VMEM((L, bw), jnp.int32),
            pltpu.VMEM((bw * L,), jnp.int32),
            pltpu.VMEM((DI * bw,), jnp.float32),
            pltpu.VMEM((DI, bw), jnp.float32),
            pltpu.VMEM((DI * VG,), jnp.float32),
            pltpu.VMEM((3, CL, DN), jnp.float32),
            pltpu.VMEM((DN, bw), jnp.float32),
            pltpu.VMEM((bw,), jnp.float32),
            pltpu.SemaphoreType.DMA,
            pltpu.SemaphoreType.DMA,
            pltpu.SemaphoreType.DMA,
            pltpu.SemaphoreType.DMA,
            pltpu.SemaphoreType.DMA,
        ],
    )
    def body(id_hbm, tokt_hbm, gics_hbm, embid_hbm, embname_hbm, embgics_hbm,
             out_hbm, id_idx, id_idx8, gics_idx, tok_t, tok_b, id_rows,
             gics_rows, gics_tab, namebuf, pooled, inv_v,
             sem_id, sem_g, sem_n0, sem_n1, sem_n2):
        wid = lax.axis_index("s") * nc + lax.axis_index("c")
        base = wid * bw

        pltpu.sync_copy(id_hbm.at[pl.ds(base, bw)], id_idx)
        pltpu.sync_copy(gics_hbm.at[pl.ds(base, bw)], gics_idx)
        pltpu.sync_copy(tokt_hbm.at[:, pl.ds(base, bw)], tok_t)

        iota = lax.iota(jnp.int32, 16)
        iota_l = iota * L
        zero16 = jnp.zeros((16,), jnp.int32)
        gpc = C // 16

        def trans_for(c):
            @plsc.parallel_loop(c * gpc, (c + 1) * gpc, unroll=2)
            def _(g):
                gbase = g * 16
                for t in range(L):
                    v = tok_t[t, pl.ds(gbase, 16)]
                    plsc.store_scatter(tok_b, [gbase * L + iota_l + t], v)

        sems = [sem_n0, sem_n1, sem_n2]

        def start_chunk(c):
            return pltpu.async_copy(
                embname_hbm.at[tok_b.at[pl.ds(c * CL, CL)]],
                namebuf.at[c % 3], sems[c % 3])

        cps = [None, None, None]
        for c in range(3):
            trans_for(c)
            cps[c] = start_chunk(c)

        @plsc.parallel_loop(0, bw // 16, unroll=2)
        def _(g):
            v = id_idx[pl.ds(g * 16, 16)]
            for d in range(DI):
                id_idx8[pl.ds(d * bw + g * 16, 16)] = v + d * VID

        cp_id = pltpu.async_copy(embid_hbm.at[id_idx8], id_rows, sem_id)
        cp_gt = pltpu.async_copy(embgics_hbm, gics_tab, sem_g)

        @plsc.parallel_loop(0, bw // 16, unroll=2)
        def _(g):
            cnt = jnp.zeros((16,), jnp.float32)
            for t in range(L):
                tk = tok_t[t, pl.ds(g * 16, 16)]
                cnt = cnt + jnp.where(tk != 0, 1.0, 0.0)
            inv_v[pl.ds(g * 16, 16)] = 1.0 / jnp.maximum(cnt, 1.0)

        def pool_chunk(c, nb):
            @plsc.parallel_loop(0, C, unroll=2)
            def _(r):
                row = c * C + r
                vals = [nb[r * L + t, :] for t in range(L)]
                while len(vals) > 1:
                    nxt = [vals[i] + vals[i + 1]
                           for i in range(0, len(vals) - 1, 2)]
                    if len(vals) % 2:
                        nxt.append(vals[-1])
                    vals = nxt
                s = plsc.load_gather(inv_v, [zero16 + row])
                plsc.store_scatter(pooled, [iota, zero16 + row], vals[0] * s)

        for c in range(nchunk):
            slot = c % 3
            cps[slot].wait()
            pool_chunk(c, namebuf.at[slot])
            if c + 3 < nchunk:
                trans_for(c + 3)
                cps[slot] = start_chunk(c + 3)

        cp_gt.wait()

        @plsc.parallel_loop(0, bw // 16, unroll=2)
        def _(g):
            gv = gics_idx[pl.ds(g * 16, 16)]
            for d in range(DI):
                r = plsc.load_gather(gics_tab, [gv + d * VG])
                gics_rows[d, pl.ds(g * 16, 16)] = r

        cp_id.wait()
        outs = [pltpu.async_copy(id_rows.at[pl.ds(d * bw, bw)],
                                 out_hbm.at[d, pl.ds(base, bw)], sem_id)
                for d in range(DI)]
        outs.append(pltpu.async_copy(
            pooled, out_hbm.at[pl.ds(DI, DN), pl.ds(base, bw)], sem_n0))
        outs.append(pltpu.async_copy(
            gics_rows, out_hbm.at[pl.ds(DI + DN, DI), pl.ds(base, bw)],
            sem_g))
        for cp in outs:
            cp.wait()

    return body


def kernel(item_id, item_name_tokens, item_gics, emb_id, emb_name, emb_gics):
    info = plsc.get_sparse_core_info()
    body = _build(info.num_cores, info.num_subcores)
    emb_name = emb_name.at[0].set(0.0)
    out_t = body(item_id.astype(jnp.int32),
                 item_name_tokens.astype(jnp.int32).T,
                 item_gics.astype(jnp.int32),
                 emb_id.T.reshape(-1), emb_name, emb_gics.T.reshape(-1))
    return out_t.T


# speedup vs baseline: 27.2521x; 1.0045x over previous
import functools

import jax
import jax.numpy as jnp
from jax import lax
from jax.experimental import pallas as pl
from jax.experimental.pallas import tpu as pltpu
from jax.experimental.pallas import tpu_sc as plsc

B = 16384
L = 20
DN = 16
DI = 8
DO = 32
VID = 100001
VG = 1001
C = 64
CL = C * L


def _build(nc, ns):
    nw = nc * ns
    bw = B // nw
    nchunk = bw // C
    mesh = plsc.VectorSubcoreMesh(core_axis_name="c", subcore_axis_name="s")

    @functools.partial(
        pl.kernel,
        out_type=jax.ShapeDtypeStruct((DO, B), jnp.float32),
        mesh=mesh,
        compiler_params=pltpu.CompilerParams(
            needs_layout_passes=False, use_tc_tiling_on_sc=False),
        scratch_types=[
            pltpu.VMEM((bw,), jnp.int32),
            pltpu.VMEM((DI * bw,), jnp.int32),
            pltpu.VMEM((bw,), jnp.int32),
            pltpu.VMEM((L, bw), jnp.int32),
            pltpu.VMEM((bw * L,), jnp.int32),
            pltpu.VMEM((DI * bw,), jnp.float32),
            pltpu.VMEM((DI, bw), jnp.float32),
            pltpu.VMEM((DI * VG,), jnp.float32),
            pltpu.VMEM((3, CL, DN), jnp.float32),
            pltpu.VMEM((DN, bw), jnp.float32),
            pltpu.VMEM((bw,), jnp.float32),
            pltpu.SemaphoreType.DMA,
            pltpu.SemaphoreType.DMA,
            pltpu.SemaphoreType.DMA,
            pltpu.SemaphoreType.DMA,
            pltpu.SemaphoreType.DMA,
        ],
    )
    def body(id_hbm, tokt_hbm, gics_hbm, embid_hbm, embname_hbm, embgics_hbm,
             out_hbm, id_idx, id_idx8, gics_idx, tok_t, tok_b, id_rows,
             gics_rows, gics_tab, namebuf, pooled, inv_v,
             sem_id, sem_g, sem_n0, sem_n1, sem_n2):
        wid = lax.axis_index("s") * nc + lax.axis_index("c")
        base = wid * bw
        import contextlib
        scope = jax.named_scope

        with scope("ph_stage"):
            pltpu.sync_copy(id_hbm.at[pl.ds(base, bw)], id_idx)
            pltpu.sync_copy(gics_hbm.at[pl.ds(base, bw)], gics_idx)
            pltpu.sync_copy(tokt_hbm.at[:, pl.ds(base, bw)], tok_t)

        iota = lax.iota(jnp.int32, 16)
        iota_l = iota * L
        zero16 = jnp.zeros((16,), jnp.int32)
        gpc = C // 16

        def trans_for(c):
            @plsc.parallel_loop(c * gpc, (c + 1) * gpc, unroll=2)
            def _(g):
                gbase = g * 16
                for t in range(L):
                    v = tok_t[t, pl.ds(gbase, 16)]
                    plsc.store_scatter(tok_b, [gbase * L + iota_l + t], v)

        sems = [sem_n0, sem_n1, sem_n2]

        def start_chunk(c):
            return pltpu.async_copy(
                embname_hbm.at[tok_b.at[pl.ds(c * CL, CL)]],
                namebuf.at[c % 3], sems[c % 3])

        cps = [None, None, None]
        with scope("ph_prime"):
            for c in range(3):
                trans_for(c)
                cps[c] = start_chunk(c)

        with scope("ph_idx8"):
            @plsc.parallel_loop(0, bw // 16, unroll=2)
            def _(g):
                v = id_idx[pl.ds(g * 16, 16)]
                for d in range(DI):
                    id_idx8[pl.ds(d * bw + g * 16, 16)] = v + d * VID

        cp_id = pltpu.async_copy(embid_hbm.at[id_idx8], id_rows, sem_id)
        cp_gt = pltpu.async_copy(embgics_hbm, gics_tab, sem_g)

        with scope("ph_cnt"):
            @plsc.parallel_loop(0, bw // 16, unroll=2)
            def _(g):
                cnt = jnp.zeros((16,), jnp.float32)
                for t in range(L):
                    tk = tok_t[t, pl.ds(g * 16, 16)]
                    cnt = cnt + jnp.where(tk != 0, 1.0, 0.0)
                inv_v[pl.ds(g * 16, 16)] = 1.0 / jnp.maximum(cnt, 1.0)

        def pool_chunk(c, nb):
            @plsc.parallel_loop(0, C, unroll=2)
            def _(r):
                row = c * C + r
                vals = [nb[r * L + t, :] for t in range(L)]
                while len(vals) > 1:
                    nxt = [vals[i] + vals[i + 1]
                           for i in range(0, len(vals) - 1, 2)]
                    if len(vals) % 2:
                        nxt.append(vals[-1])
                    vals = nxt
                s = plsc.load_gather(inv_v, [zero16 + row])
                plsc.store_scatter(pooled, [iota, zero16 + row], vals[0] * s)

        for c in range(nchunk):
            slot = c % 3
            with scope(f"ph_wait{c}"):
                cps[slot].wait()
            with scope(f"ph_pool{c}"):
                pool_chunk(c, namebuf.at[slot])
                if c + 3 < nchunk:
                    trans_for(c + 3)
                    cps[slot] = start_chunk(c + 3)

        cp_gt.wait()

        with scope("ph_gics"):
            @plsc.parallel_loop(0, bw // 16, unroll=2)
            def _(g):
                gv = gics_idx[pl.ds(g * 16, 16)]
                for d in range(DI):
                    r = plsc.load_gather(gics_tab, [gv + d * VG])
                    gics_rows[d, pl.ds(g * 16, 16)] = r

        with scope("ph_waitid"):
            cp_id.wait()
        outs = [pltpu.async_copy(id_rows.at[pl.ds(d * bw, bw)],
                                 out_hbm.at[d, pl.ds(base, bw)], sem_id)
                for d in range(DI)]
        outs.append(pltpu.async_copy(
            pooled, out_hbm.at[pl.ds(DI, DN), pl.ds(base, bw)], sem_n0))
        outs.append(pltpu.async_copy(
            gics_rows, out_hbm.at[pl.ds(DI + DN, DI), pl.ds(base, bw)],
            sem_g))
        with scope("ph_out"):
            for cp in outs:
                cp.wait()

    return body


def kernel(item_id, item_name_tokens, item_gics, emb_id, emb_name, emb_gics):
    info = plsc.get_sparse_core_info()
    body = _build(info.num_cores, info.num_subcores)
    emb_name = emb_name.at[0].set(0.0)
    out_t = body(item_id.astype(jnp.int32),
                 item_name_tokens.astype(jnp.int32).T,
                 item_gics.astype(jnp.int32),
                 emb_id.T.reshape(-1), emb_name, emb_gics.T.reshape(-1))
    return out_t.T


# speedup vs baseline: 27.3422x; 1.0033x over previous
import functools

import jax
import jax.numpy as jnp
from jax import lax
from jax.experimental import pallas as pl
from jax.experimental.pallas import tpu as pltpu
from jax.experimental.pallas import tpu_sc as plsc

B = 16384
L = 20
DN = 16
DI = 8
DO = 32
VID = 100001
VG = 1001
C = 64
CL = C * L


def _build(nc, ns):
    nw = nc * ns
    bw = B // nw
    nchunk = bw // C
    mesh = plsc.VectorSubcoreMesh(core_axis_name="c", subcore_axis_name="s")

    @functools.partial(
        pl.kernel,
        out_type=jax.ShapeDtypeStruct((DO, B), jnp.float32),
        mesh=mesh,
        compiler_params=pltpu.CompilerParams(
            needs_layout_passes=False, use_tc_tiling_on_sc=False),
        scratch_types=[
            pltpu.VMEM((bw,), jnp.int32),
            pltpu.VMEM((DI * bw,), jnp.int32),
            pltpu.VMEM((bw,), jnp.int32),
            pltpu.VMEM((L, bw), jnp.int32),
            pltpu.VMEM((bw * L,), jnp.int32),
            pltpu.VMEM((DI * bw,), jnp.float32),
            pltpu.VMEM((DI, bw), jnp.float32),
            pltpu.VMEM((DI * VG,), jnp.float32),
            pltpu.VMEM((3, CL, DN), jnp.float32),
            pltpu.VMEM((DN, bw), jnp.float32),
            pltpu.VMEM((bw,), jnp.float32),
            pltpu.SemaphoreType.DMA,
            pltpu.SemaphoreType.DMA,
            pltpu.SemaphoreType.DMA,
            pltpu.SemaphoreType.DMA,
            pltpu.SemaphoreType.DMA,
        ],
    )
    def body(id_hbm, tokt_hbm, gics_hbm, embid_hbm, embname_hbm, embgics_hbm,
             out_hbm, id_idx, id_idx8, gics_idx, tok_t, tok_b, id_rows,
             gics_rows, gics_tab, namebuf, pooled, inv_v,
             sem_id, sem_g, sem_n0, sem_n1, sem_n2):
        wid = lax.axis_index("s") * nc + lax.axis_index("c")
        base = wid * bw

        cp_tok = pltpu.async_copy(tokt_hbm.at[:, pl.ds(base, bw)], tok_t,
                                  sem_n0)
        cp_iidx = pltpu.async_copy(id_hbm.at[pl.ds(base, bw)], id_idx,
                                   sem_id)
        cp_gidx = pltpu.async_copy(gics_hbm.at[pl.ds(base, bw)], gics_idx,
                                   sem_g)
        cp_tok.wait()

        iota = lax.iota(jnp.int32, 16)
        iota_l = iota * L
        zero16 = jnp.zeros((16,), jnp.int32)

        chunks = [(0, 16), (16, 16), (32, 32), (64, 64), (128, 64),
                  (192, 64), (256, 64), (320, 64), (384, 64), (448, 64)]
        nck = len(chunks)

        def trans_for(ci):
            s, n = chunks[ci]

            @plsc.parallel_loop(s // 16, (s + n) // 16, unroll=2)
            def _(g):
                gbase = g * 16
                for t in range(L):
                    v = tok_t[t, pl.ds(gbase, 16)]
                    plsc.store_scatter(tok_b, [gbase * L + iota_l + t], v)

        sems = [sem_n0, sem_n1, sem_n2]

        def start_chunk(ci):
            s, n = chunks[ci]
            return pltpu.async_copy(
                embname_hbm.at[tok_b.at[pl.ds(s * L, n * L)]],
                namebuf.at[ci % 3].at[pl.ds(0, n * L)], sems[ci % 3])

        cps = [None, None, None]
        for ci in range(3):
            trans_for(ci)
            cps[ci] = start_chunk(ci)

        @plsc.parallel_loop(0, bw // 16, unroll=2)
        def _(g):
            cnt = jnp.zeros((16,), jnp.float32)
            for t in range(L):
                tk = tok_t[t, pl.ds(g * 16, 16)]
                cnt = cnt + jnp.where(tk != 0, 1.0, 0.0)
            inv_v[pl.ds(g * 16, 16)] = 1.0 / jnp.maximum(cnt, 1.0)

        def pool_chunk(ci, nb):
            s, n = chunks[ci]

            @plsc.parallel_loop(0, n, unroll=2)
            def _(r):
                row = s + r
                vals = [nb[r * L + t, :] for t in range(L)]
                while len(vals) > 1:
                    nxt = [vals[i] + vals[i + 1]
                           for i in range(0, len(vals) - 1, 2)]
                    if len(vals) % 2:
                        nxt.append(vals[-1])
                    vals = nxt
                s_ = plsc.load_gather(inv_v, [zero16 + row])
                plsc.store_scatter(pooled, [iota, zero16 + row], vals[0] * s_)

        cp_id = None
        cp_gt = None
        for ci in range(nck):
            slot = ci % 3
            cps[slot].wait()
            pool_chunk(ci, namebuf.at[slot])
            if ci + 3 < nck:
                trans_for(ci + 3)
                cps[slot] = start_chunk(ci + 3)
            if ci == 2:
                cp_iidx.wait()

                @plsc.parallel_loop(0, bw // 16, unroll=2)
                def _(g):
                    v = id_idx[pl.ds(g * 16, 16)]
                    for d in range(DI):
                        id_idx8[pl.ds(d * bw + g * 16, 16)] = v + d * VID

                cp_id = pltpu.async_copy(embid_hbm.at[id_idx8], id_rows,
                                         sem_id)
                cp_gt = pltpu.async_copy(embgics_hbm, gics_tab, sem_g)

        cp_gidx.wait()
        cp_gt.wait()

        @plsc.parallel_loop(0, bw // 16, unroll=2)
        def _(g):
            gv = gics_idx[pl.ds(g * 16, 16)]
            for d in range(DI):
                r = plsc.load_gather(gics_tab, [gv + d * VG])
                gics_rows[d, pl.ds(g * 16, 16)] = r

        cp_id.wait()
        outs = [pltpu.async_copy(id_rows.at[pl.ds(d * bw, bw)],
                                 out_hbm.at[d, pl.ds(base, bw)], sem_id)
                for d in range(DI)]
        outs.append(pltpu.async_copy(
            pooled, out_hbm.at[pl.ds(DI, DN), pl.ds(base, bw)], sem_n0))
        outs.append(pltpu.async_copy(
            gics_rows, out_hbm.at[pl.ds(DI + DN, DI), pl.ds(base, bw)],
            sem_g))
        for cp in outs:
            cp.wait()

    return body


def kernel(item_id, item_name_tokens, item_gics, emb_id, emb_name, emb_gics):
    info = plsc.get_sparse_core_info()
    body = _build(info.num_cores, info.num_subcores)
    emb_name = emb_name.at[0].set(0.0)
    out_t = body(item_id.astype(jnp.int32),
                 item_name_tokens.astype(jnp.int32).T,
                 item_gics.astype(jnp.int32),
                 emb_id.T.reshape(-1), emb_name, emb_gics.T.reshape(-1))
    return out_t.T
